# Initial kernel scaffold; baseline (speedup 1.0000x reference)
import jax
import jax.numpy as jnp
from jax.experimental import pallas as pl


def kernel(h, edge_index, e, snorm_n, W_pre, b_pre, W_post, b_post):
    raise NotImplementedError("write your pallas kernel here")


# speedup vs baseline: 1.9420x; 1.9420x over previous
import functools
import jax
import jax.numpy as jnp
from jax import lax
from jax.experimental import pallas as pl
from jax.experimental.pallas import tpu as pltpu
from jax.experimental.pallas import tpu_sc as plsc

N = 10000
E = 320000
D = 128
D_EDGE = 16
AVG_D_LOG = 3.4965075614664802
EPS = 1e-5

NC = 2
NS = 16
NW = NC * NS
NPASS = 2
NG = NW * NPASS
NPG = 160
NPAD = NG * NPG
CHUNK = 2000
NITER = CHUNK // 16
NCHUNK = E // CHUNK
FLT_MAX = 3.4028235e38


def _pre_node_body(h_ref, w_ref, o_ref):
    o_ref[...] = jnp.dot(h_ref[...], w_ref[...],
                         preferred_element_type=jnp.float32)


def _pre_edge_body(e_ref, w_ref, b_ref, o_ref):
    o_ref[...] = jnp.dot(e_ref[...], w_ref[...],
                         preferred_element_type=jnp.float32) + b_ref[...]


def _tc_pre(h, e, Wps, Wpe, b_pre):
    A = pl.pallas_call(
        _pre_node_body,
        grid=(5,),
        in_specs=[pl.BlockSpec((2000, D), lambda i: (i, 0)),
                  pl.BlockSpec((D, D), lambda i: (0, 0))],
        out_specs=pl.BlockSpec((2000, D), lambda i: (i, 0)),
        out_shape=jax.ShapeDtypeStruct((N, D), jnp.float32),
    )(h, Wps)
    C = pl.pallas_call(
        _pre_edge_body,
        grid=(160,),
        in_specs=[pl.BlockSpec((CHUNK, D_EDGE), lambda i: (i, 0)),
                  pl.BlockSpec((D_EDGE, D), lambda i: (0, 0)),
                  pl.BlockSpec((1, D), lambda i: (0, 0))],
        out_specs=pl.BlockSpec((CHUNK, D), lambda i: (i, 0)),
        out_shape=jax.ShapeDtypeStruct((E, D), jnp.float32),
    )(e, Wpe, b_pre.reshape(1, D))
    return A, C


def _sc_body(A_hbm, C_hbm, dst_hbm, src_hbm,
             o_sum, o_sq, o_mx, o_mn, o_deg,
             acc_sum, acc_sq, acc_mx, acc_mn, acc_deg,
             dbuf, sbuf, pend_d, pend_s, pend_e,
             gidx_s, gidx_e, astage, cstage, sem_a, sem_c):
    c = lax.axis_index("c")
    s = lax.axis_index("s")
    tid = s * NC + c
    lane = lax.iota(jnp.int32, 16)
    zeros16 = jnp.zeros((16,), jnp.float32)
    izeros16 = jnp.zeros((16,), jnp.int32)

    def process_edges(n_edges):
        gidx_s[...] = pend_s[pl.ds(0, 16)]
        gidx_e[...] = pend_e[pl.ds(0, 16)]
        cp_a = pltpu.async_copy(A_hbm.at[gidx_s], astage, sem_a)
        cp_c = pltpu.async_copy(C_hbm.at[gidx_e], cstage, sem_c)
        cp_a.wait()
        cp_c.wait()
        dvec = pend_d[pl.ds(0, 16)]

        def edge_body(k, _):
            dnums = lax.GatherDimensionNumbers(
                offset_dims=(), collapsed_slice_dims=(0,),
                start_index_map=(0,))
            dl = lax.gather(dvec, jnp.full((16, 1), 0, jnp.int32) + k,
                            dnums, (1,),
                            mode=lax.GatherScatterMode.PROMISE_IN_BOUNDS)[0]
            for j in range(8):
                sl = pl.ds(j * 16, 16)
                x = astage[k, sl] + cstage[k, sl]
                acc_sum[dl, sl] = acc_sum[dl, sl] + x
                acc_sq[dl, sl] = acc_sq[dl, sl] + x * x
                acc_mx[dl, sl] = jnp.maximum(acc_mx[dl, sl], x)
                acc_mn[dl, sl] = jnp.minimum(acc_mn[dl, sl], x)
            dbase = (dl // 16) * 16
            dwin = acc_deg[pl.ds(dbase, 16)]
            acc_deg[pl.ds(dbase, 16)] = dwin + jnp.where(
                lane == dl - dbase, 1.0, 0.0)
            return 0

        lax.fori_loop(0, n_edges, edge_body, 0)

    for p in range(NPASS):
        g = p * NW + tid
        glo = g * NPG

        def init_body(r, _):
            for j in range(8):
                sl = pl.ds(j * 16, 16)
                acc_sum[r, sl] = zeros16
                acc_sq[r, sl] = zeros16
                acc_mx[r, sl] = zeros16 - FLT_MAX
                acc_mn[r, sl] = zeros16 + FLT_MAX
            return 0
        lax.fori_loop(0, NPG, init_body, 0)
        def dinit_body(r, _):
            acc_deg[pl.ds(r * 16, 16)] = zeros16
            return 0
        lax.fori_loop(0, NPG // 16, dinit_body, 0)

        for half in (0, 16):
            pend_d[pl.ds(half, 16)] = izeros16
            pend_s[pl.ds(half, 16)] = izeros16
            pend_e[pl.ds(half, 16)] = izeros16

        def chunk_body(ci, cnt):
            off = ci * CHUNK
            pltpu.sync_copy(dst_hbm.at[pl.ds(off, CHUNK)], dbuf)
            pltpu.sync_copy(src_hbm.at[pl.ds(off, CHUNK)], sbuf)

            def filt_body(i, cnt):
                dv = dbuf[pl.ds(i * 16, 16)]
                m = (dv >= glo) & (dv < glo + NPG)
                nm = plsc.all_reduce_population_count(m)[0]
                plsc.store_compressed(pend_d.at[pl.ds(cnt, 16)],
                                      dv - glo, mask=m)
                plsc.store_compressed(pend_s.at[pl.ds(cnt, 16)],
                                      sbuf[pl.ds(i * 16, 16)], mask=m)
                plsc.store_compressed(pend_e.at[pl.ds(cnt, 16)],
                                      off + i * 16 + lane, mask=m)
                cnt = cnt + nm
                do_flush = cnt >= 16

                @pl.when(do_flush)
                def _():
                    process_edges(16)
                    pend_d[pl.ds(0, 16)] = pend_d[pl.ds(16, 16)]
                    pend_s[pl.ds(0, 16)] = pend_s[pl.ds(16, 16)]
                    pend_e[pl.ds(0, 16)] = pend_e[pl.ds(16, 16)]

                return jnp.where(do_flush, cnt - 16, cnt)

            return lax.fori_loop(0, NITER, filt_body, cnt)

        cnt = lax.fori_loop(0, NCHUNK, chunk_body, jnp.int32(0))

        @pl.when(cnt > 0)
        def _():
            process_edges(cnt)

        pltpu.sync_copy(acc_sum, o_sum.at[pl.ds(glo, NPG)])
        pltpu.sync_copy(acc_sq, o_sq.at[pl.ds(glo, NPG)])
        pltpu.sync_copy(acc_mx, o_mx.at[pl.ds(glo, NPG)])
        pltpu.sync_copy(acc_mn, o_mn.at[pl.ds(glo, NPG)])
        pltpu.sync_copy(acc_deg.at[pl.ds(0, NPG)], o_deg.at[pl.ds(glo, NPG)])


def _sc_aggregate(A, C, src, dst):
    f32 = jnp.float32
    out_type = (jax.ShapeDtypeStruct((NPAD, D), f32),
                jax.ShapeDtypeStruct((NPAD, D), f32),
                jax.ShapeDtypeStruct((NPAD, D), f32),
                jax.ShapeDtypeStruct((NPAD, D), f32),
                jax.ShapeDtypeStruct((NPAD,), f32))
    scratch = [
        pltpu.VMEM((NPG, D), f32),
        pltpu.VMEM((NPG, D), f32),
        pltpu.VMEM((NPG, D), f32),
        pltpu.VMEM((NPG, D), f32),
        pltpu.VMEM((NPG,), f32),
        pltpu.VMEM((CHUNK,), jnp.int32),
        pltpu.VMEM((CHUNK,), jnp.int32),
        pltpu.VMEM((32,), jnp.int32),
        pltpu.VMEM((32,), jnp.int32),
        pltpu.VMEM((32,), jnp.int32),
        pltpu.VMEM((16,), jnp.int32),
        pltpu.VMEM((16,), jnp.int32),
        pltpu.VMEM((16, D), f32),
        pltpu.VMEM((16, D), f32),
        pltpu.SemaphoreType.DMA,
        pltpu.SemaphoreType.DMA,
    ]
    mesh = plsc.VectorSubcoreMesh(core_axis_name="c", subcore_axis_name="s",
                                  num_cores=NC, num_subcores=NS)
    fn = pl.kernel(_sc_body, out_type=out_type, mesh=mesh,
                   scratch_types=scratch,
                   compiler_params=pltpu.CompilerParams(
                       needs_layout_passes=False))
    return fn(A, C, dst, src)


def _post_body(h_ref, s1_ref, s2_ref, mx_ref, mn_ref, deg_ref, sn_ref,
               wpd_ref, wh_ref, wa_ref, wb_ref, wc_ref, bp_ref, o_ref):
    f32 = jnp.float32
    h = h_ref[...]
    B = jnp.dot(h, wpd_ref[...], preferred_element_type=f32)
    deg = deg_ref[...]
    s1 = s1_ref[...]
    s2 = s2_ref[...]
    degc = jnp.maximum(deg, 1.0)
    has = deg > 0.0
    mean = (s1 + deg * B) / degc
    mx = jnp.where(has, mx_ref[...] + B, 0.0)
    mn = jnp.where(has, mn_ref[...] + B, 0.0)
    mean_sq = (s2 + 2.0 * B * s1 + deg * B * B) / degc
    var = jnp.maximum(mean_sq - mean * mean, 0.0)
    std = jnp.sqrt(var + EPS)
    agg = jnp.concatenate([mean, mx, mn, std], axis=1)
    logd = jnp.log(degc + 1.0)
    amp = logd * (1.0 / AVG_D_LOG)
    att = AVG_D_LOG / logd
    acc = (jnp.dot(h, wh_ref[...], preferred_element_type=f32)
           + jnp.dot(agg, wa_ref[...], preferred_element_type=f32)
           + jnp.dot(agg * amp, wb_ref[...], preferred_element_type=f32)
           + jnp.dot(agg * att, wc_ref[...], preferred_element_type=f32))
    o_ref[...] = (acc + bp_ref[...]) * sn_ref[...]


def _tc_post(h, s1, s2, mx, mn, deg, snorm, Wpd, Wh, Wa, Wb, Wc, b_post):
    R = 2000
    node_spec = pl.BlockSpec((R, D), lambda i: (i, 0))
    col_spec = pl.BlockSpec((R, 1), lambda i: (i, 0))
    full = lambda r, c: pl.BlockSpec((r, c), lambda i: (0, 0))
    return pl.pallas_call(
        _post_body,
        grid=(5,),
        in_specs=[node_spec, node_spec, node_spec, node_spec, node_spec,
                  col_spec, col_spec,
                  full(D, D), full(D, D), full(512, D), full(512, D),
                  full(512, D), full(1, D)],
        out_specs=node_spec,
        out_shape=jax.ShapeDtypeStruct((N, D), jnp.float32),
    )(h, s1, s2, mx, mn, deg, snorm, Wpd, Wh, Wa, Wb, Wc, b_post)


def kernel(h, edge_index, e, snorm_n, W_pre, b_pre, W_post, b_post):
    Wps = W_pre[0:D]
    Wpd = W_pre[D:2 * D]
    Wpe = W_pre[2 * D:]
    Wh = W_post[0:D]
    Wa = W_post[D:D + 512]
    Wb = W_post[D + 512:D + 1024]
    Wc = W_post[D + 1024:]
    src = edge_index[0]
    dst = edge_index[1]

    A, C = _tc_pre(h, e, Wps, Wpe, b_pre)
    s1, s2, mx, mn, deg = _sc_aggregate(A, C, src, dst)
    out = _tc_post(h, s1[:N], s2[:N], mx[:N], mn[:N],
                   deg[:N].reshape(N, 1), snorm_n,
                   Wpd, Wh, Wa, Wb, Wc, b_post.reshape(1, D))
    return out


# speedup vs baseline: 2.0558x; 1.0586x over previous
import functools
import jax
import jax.numpy as jnp
from jax import lax
from jax.experimental import pallas as pl
from jax.experimental.pallas import tpu as pltpu
from jax.experimental.pallas import tpu_sc as plsc

N = 10000
E = 320000
D = 128
D_EDGE = 16
AVG_D_LOG = 3.4965075614664802
EPS = 1e-5

NC = 2
NS = 16
NW = NC * NS
NPASS = 2
NG = NW * NPASS
NPG = 160
NPAD = NG * NPG
CHUNK = 2000
NITER = CHUNK // 16
NCHUNK = E // CHUNK
FLT_MAX = 3.4028235e38
FK = 64


def _pre_node_body(h_ref, w_ref, o_ref):
    o_ref[...] = jnp.dot(h_ref[...], w_ref[...],
                         preferred_element_type=jnp.float32)


def _pre_edge_body(e_ref, w_ref, b_ref, o_ref):
    o_ref[...] = jnp.dot(e_ref[...], w_ref[...],
                         preferred_element_type=jnp.float32) + b_ref[...]


def _tc_pre(h, e, Wps, Wpe, b_pre):
    A = pl.pallas_call(
        _pre_node_body,
        grid=(5,),
        in_specs=[pl.BlockSpec((2000, D), lambda i: (i, 0)),
                  pl.BlockSpec((D, D), lambda i: (0, 0))],
        out_specs=pl.BlockSpec((2000, D), lambda i: (i, 0)),
        out_shape=jax.ShapeDtypeStruct((N, D), jnp.float32),
    )(h, Wps)
    C = pl.pallas_call(
        _pre_edge_body,
        grid=(160,),
        in_specs=[pl.BlockSpec((CHUNK, D_EDGE), lambda i: (i, 0)),
                  pl.BlockSpec((D_EDGE, D), lambda i: (0, 0)),
                  pl.BlockSpec((1, D), lambda i: (0, 0))],
        out_specs=pl.BlockSpec((CHUNK, D), lambda i: (i, 0)),
        out_shape=jax.ShapeDtypeStruct((E, D), jnp.float32),
    )(e, Wpe, b_pre.reshape(1, D))
    return A, C


def _sc_body(A_hbm, C_hbm, dst_hbm, src_hbm,
             o_sum, o_sq, o_mx, o_mn, o_deg,
             acc_sum, acc_sq, acc_mx, acc_mn, acc_deg,
             dbuf, sbuf, pend_d, pend_s, pend_e,
             gidx_s, gidx_e, astage, cstage, sem_a, sem_c):
    c = lax.axis_index("c")
    s = lax.axis_index("s")
    tid = s * NC + c
    lane = lax.iota(jnp.int32, 16)
    zeros16 = jnp.zeros((16,), jnp.float32)
    izeros16 = jnp.zeros((16,), jnp.int32)

    def process_edges(n_edges):
        for q in range(FK // 16):
            gidx_s[pl.ds(q * 16, 16)] = pend_s[pl.ds(q * 16, 16)]
            gidx_e[pl.ds(q * 16, 16)] = pend_e[pl.ds(q * 16, 16)]
        cp_a = pltpu.async_copy(A_hbm.at[gidx_s], astage, sem_a)
        cp_c = pltpu.async_copy(C_hbm.at[gidx_e], cstage, sem_c)
        cp_a.wait()
        cp_c.wait()

        def edge_body(k, _):
            gbase = (k // 16) * 16
            dvec = pend_d[pl.ds(gbase, 16)]
            dnums = lax.GatherDimensionNumbers(
                offset_dims=(), collapsed_slice_dims=(0,),
                start_index_map=(0,))
            dl = lax.gather(dvec, jnp.full((16, 1), 0, jnp.int32)
                            + (k - gbase), dnums, (1,),
                            mode=lax.GatherScatterMode.PROMISE_IN_BOUNDS)[0]
            for j in range(8):
                sl = pl.ds(j * 16, 16)
                x = astage[k, sl] + cstage[k, sl]
                acc_sum[dl, sl] = acc_sum[dl, sl] + x
                acc_sq[dl, sl] = acc_sq[dl, sl] + x * x
                acc_mx[dl, sl] = jnp.maximum(acc_mx[dl, sl], x)
                acc_mn[dl, sl] = jnp.minimum(acc_mn[dl, sl], x)
            dbase = (dl // 16) * 16
            dwin = acc_deg[pl.ds(dbase, 16)]
            acc_deg[pl.ds(dbase, 16)] = dwin + jnp.where(
                lane == dl - dbase, 1.0, 0.0)
            return 0

        lax.fori_loop(0, n_edges, edge_body, 0)

    for p in range(NPASS):
        g = p * NW + tid
        glo = g * NPG

        def init_body(r, _):
            for j in range(8):
                sl = pl.ds(j * 16, 16)
                acc_sum[r, sl] = zeros16
                acc_sq[r, sl] = zeros16
                acc_mx[r, sl] = zeros16 - FLT_MAX
                acc_mn[r, sl] = zeros16 + FLT_MAX
            return 0
        lax.fori_loop(0, NPG, init_body, 0)
        def dinit_body(r, _):
            acc_deg[pl.ds(r * 16, 16)] = zeros16
            return 0
        lax.fori_loop(0, NPG // 16, dinit_body, 0)

        for half in range(0, FK + 16, 16):
            pend_d[pl.ds(half, 16)] = izeros16
            pend_s[pl.ds(half, 16)] = izeros16
            pend_e[pl.ds(half, 16)] = izeros16

        def chunk_body(ci, cnt):
            off = ci * CHUNK
            pltpu.sync_copy(dst_hbm.at[pl.ds(off, CHUNK)], dbuf)
            pltpu.sync_copy(src_hbm.at[pl.ds(off, CHUNK)], sbuf)

            def filt_body(i, cnt):
                dv = dbuf[pl.ds(i * 16, 16)]
                m = (dv >= glo) & (dv < glo + NPG)
                nm = plsc.all_reduce_population_count(m)[0]
                plsc.store_compressed(pend_d.at[pl.ds(cnt, 16)],
                                      dv - glo, mask=m)
                plsc.store_compressed(pend_s.at[pl.ds(cnt, 16)],
                                      sbuf[pl.ds(i * 16, 16)], mask=m)
                plsc.store_compressed(pend_e.at[pl.ds(cnt, 16)],
                                      off + i * 16 + lane, mask=m)
                cnt = cnt + nm
                do_flush = cnt >= FK

                @pl.when(do_flush)
                def _():
                    process_edges(FK)
                    pend_d[pl.ds(0, 16)] = pend_d[pl.ds(FK, 16)]
                    pend_s[pl.ds(0, 16)] = pend_s[pl.ds(FK, 16)]
                    pend_e[pl.ds(0, 16)] = pend_e[pl.ds(FK, 16)]

                return jnp.where(do_flush, cnt - FK, cnt)

            return lax.fori_loop(0, NITER, filt_body, cnt)

        cnt = lax.fori_loop(0, NCHUNK, chunk_body, jnp.int32(0))

        @pl.when(cnt > 0)
        def _():
            process_edges(cnt)

        pltpu.sync_copy(acc_sum, o_sum.at[pl.ds(glo, NPG)])
        pltpu.sync_copy(acc_sq, o_sq.at[pl.ds(glo, NPG)])
        pltpu.sync_copy(acc_mx, o_mx.at[pl.ds(glo, NPG)])
        pltpu.sync_copy(acc_mn, o_mn.at[pl.ds(glo, NPG)])
        pltpu.sync_copy(acc_deg.at[pl.ds(0, NPG)], o_deg.at[pl.ds(glo, NPG)])


def _sc_aggregate(A, C, src, dst):
    f32 = jnp.float32
    out_type = (jax.ShapeDtypeStruct((NPAD, D), f32),
                jax.ShapeDtypeStruct((NPAD, D), f32),
                jax.ShapeDtypeStruct((NPAD, D), f32),
                jax.ShapeDtypeStruct((NPAD, D), f32),
                jax.ShapeDtypeStruct((NPAD,), f32))
    scratch = [
        pltpu.VMEM((NPG, D), f32),
        pltpu.VMEM((NPG, D), f32),
        pltpu.VMEM((NPG, D), f32),
        pltpu.VMEM((NPG, D), f32),
        pltpu.VMEM((NPG,), f32),
        pltpu.VMEM((CHUNK,), jnp.int32),
        pltpu.VMEM((CHUNK,), jnp.int32),
        pltpu.VMEM((FK + 16,), jnp.int32),
        pltpu.VMEM((FK + 16,), jnp.int32),
        pltpu.VMEM((FK + 16,), jnp.int32),
        pltpu.VMEM((FK,), jnp.int32),
        pltpu.VMEM((FK,), jnp.int32),
        pltpu.VMEM((FK, D), f32),
        pltpu.VMEM((FK, D), f32),
        pltpu.SemaphoreType.DMA,
        pltpu.SemaphoreType.DMA,
    ]
    mesh = plsc.VectorSubcoreMesh(core_axis_name="c", subcore_axis_name="s",
                                  num_cores=NC, num_subcores=NS)
    fn = pl.kernel(_sc_body, out_type=out_type, mesh=mesh,
                   scratch_types=scratch,
                   compiler_params=pltpu.CompilerParams(
                       needs_layout_passes=False))
    return fn(A, C, dst, src)


def _post_body(h_ref, s1_ref, s2_ref, mx_ref, mn_ref, deg_ref, sn_ref,
               wpd_ref, wh_ref, wa_ref, wb_ref, wc_ref, bp_ref, o_ref):
    f32 = jnp.float32
    h = h_ref[...]
    B = jnp.dot(h, wpd_ref[...], preferred_element_type=f32)
    deg = deg_ref[...]
    s1 = s1_ref[...]
    s2 = s2_ref[...]
    degc = jnp.maximum(deg, 1.0)
    has = deg > 0.0
    mean = (s1 + deg * B) / degc
    mx = jnp.where(has, mx_ref[...] + B, 0.0)
    mn = jnp.where(has, mn_ref[...] + B, 0.0)
    mean_sq = (s2 + 2.0 * B * s1 + deg * B * B) / degc
    var = jnp.maximum(mean_sq - mean * mean, 0.0)
    std = jnp.sqrt(var + EPS)
    agg = jnp.concatenate([mean, mx, mn, std], axis=1)
    logd = jnp.log(degc + 1.0)
    amp = logd * (1.0 / AVG_D_LOG)
    att = AVG_D_LOG / logd
    acc = (jnp.dot(h, wh_ref[...], preferred_element_type=f32)
           + jnp.dot(agg, wa_ref[...], preferred_element_type=f32)
           + jnp.dot(agg * amp, wb_ref[...], preferred_element_type=f32)
           + jnp.dot(agg * att, wc_ref[...], preferred_element_type=f32))
    o_ref[...] = (acc + bp_ref[...]) * sn_ref[...]


def _tc_post(h, s1, s2, mx, mn, deg, snorm, Wpd, Wh, Wa, Wb, Wc, b_post):
    R = 2000
    node_spec = pl.BlockSpec((R, D), lambda i: (i, 0))
    col_spec = pl.BlockSpec((R, 1), lambda i: (i, 0))
    full = lambda r, c: pl.BlockSpec((r, c), lambda i: (0, 0))
    return pl.pallas_call(
        _post_body,
        grid=(5,),
        in_specs=[node_spec, node_spec, node_spec, node_spec, node_spec,
                  col_spec, col_spec,
                  full(D, D), full(D, D), full(512, D), full(512, D),
                  full(512, D), full(1, D)],
        out_specs=node_spec,
        out_shape=jax.ShapeDtypeStruct((N, D), jnp.float32),
    )(h, s1, s2, mx, mn, deg, snorm, Wpd, Wh, Wa, Wb, Wc, b_post)


def kernel(h, edge_index, e, snorm_n, W_pre, b_pre, W_post, b_post):
    Wps = W_pre[0:D]
    Wpd = W_pre[D:2 * D]
    Wpe = W_pre[2 * D:]
    Wh = W_post[0:D]
    Wa = W_post[D:D + 512]
    Wb = W_post[D + 512:D + 1024]
    Wc = W_post[D + 1024:]
    src = edge_index[0]
    dst = edge_index[1]

    A, C = _tc_pre(h, e, Wps, Wpe, b_pre)
    s1, s2, mx, mn, deg = _sc_aggregate(A, C, src, dst)
    out = _tc_post(h, s1[:N], s2[:N], mx[:N], mn[:N],
                   deg[:N].reshape(N, 1), snorm_n,
                   Wpd, Wh, Wa, Wb, Wc, b_post.reshape(1, D))
    return out


# speedup vs baseline: 2.0727x; 1.0082x over previous
import jax
import jax.numpy as jnp
from jax import lax
from jax.experimental import pallas as pl
from jax.experimental.pallas import tpu as pltpu
from jax.experimental.pallas import tpu_sc as plsc

N = 10000
E = 320000
D = 128
D_EDGE = 16
AVG_D_LOG = 3.4965075614664802
EPS = 1e-5

NC = 2
NS = 16
NW = NC * NS
NPASS = 2
NG = NW * NPASS
NPG = 160
NPAD = NG * NPG
NPSC = NS * NPG
DUMMY = NPSC
CHUNK = 2000
NITER = CHUNK // 16
NCHUNK = E // CHUNK
FLT_MAX = 3.4028235e38
FK = 64


def _pre_node_body(h_ref, w_ref, o_ref):
    o_ref[...] = jnp.dot(h_ref[...], w_ref[...],
                         preferred_element_type=jnp.float32)


def _pre_edge_body(e_ref, w_ref, b_ref, o_ref):
    o_ref[...] = jnp.dot(e_ref[...], w_ref[...],
                         preferred_element_type=jnp.float32) + b_ref[...]


def _tc_pre(h, e, Wps, Wpe, b_pre):
    A = pl.pallas_call(
        _pre_node_body,
        grid=(5,),
        in_specs=[pl.BlockSpec((2000, D), lambda i: (i, 0)),
                  pl.BlockSpec((D, D), lambda i: (0, 0))],
        out_specs=pl.BlockSpec((2000, D), lambda i: (i, 0)),
        out_shape=jax.ShapeDtypeStruct((N, D), jnp.float32),
    )(h, Wps)
    C = pl.pallas_call(
        _pre_edge_body,
        grid=(160,),
        in_specs=[pl.BlockSpec((CHUNK, D_EDGE), lambda i: (i, 0)),
                  pl.BlockSpec((D_EDGE, D), lambda i: (0, 0)),
                  pl.BlockSpec((1, D), lambda i: (0, 0))],
        out_specs=pl.BlockSpec((CHUNK, D), lambda i: (i, 0)),
        out_shape=jax.ShapeDtypeStruct((E, D), jnp.float32),
    )(e, Wpe, b_pre.reshape(1, D))
    return A, C


def _sc_body(A_hbm, C_hbm, dst_hbm, src_hbm,
             o_sum, o_sq, o_mx, o_mn, o_deg,
             ssum, ssq,
             acc_mx, acc_mn, acc_deg,
             dbuf0, sbuf0, dbuf1, sbuf1,
             pend_d, pend_s, pend_e,
             gidx_d, gidx_s, gidx_e,
             astage, cstage, xbuf, x2buf,
             sem_a, sem_c, semd0, sems0, semd1, sems1):
    c = lax.axis_index("c")
    s = lax.axis_index("s")
    lane = lax.iota(jnp.int32, 16)
    zeros16 = jnp.zeros((16,), jnp.float32)
    izeros16 = jnp.zeros((16,), jnp.int32)
    tlo = s * NPG

    def run_pass(glo, sclo):

        def process_edges(n_edges):
            for q in range(FK // 16):
                sl16 = pl.ds(q * 16, 16)
                gidx_s[sl16] = pend_s[sl16]
                gidx_e[sl16] = pend_e[sl16]
                active = (q * 16 + lane) < n_edges
                gidx_d[sl16] = jnp.where(active, pend_d[sl16], DUMMY)
            cp_a = pltpu.async_copy(A_hbm.at[gidx_s], astage, sem_a)
            cp_c = pltpu.async_copy(C_hbm.at[gidx_e], cstage, sem_c)
            cp_a.wait()
            cp_c.wait()

            def edge_body(k, _):
                gbase = (k // 16) * 16
                dvec = pend_d[pl.ds(gbase, 16)]
                dnums = lax.GatherDimensionNumbers(
                    offset_dims=(), collapsed_slice_dims=(0,),
                    start_index_map=(0,))
                dl = lax.gather(dvec, jnp.full((16, 1), 0, jnp.int32)
                                + (k - gbase), dnums, (1,),
                                mode=lax.GatherScatterMode.PROMISE_IN_BOUNDS,
                                )[0]
                dt = dl - tlo
                for j in range(8):
                    sl = pl.ds(j * 16, 16)
                    x = astage[k, sl] + cstage[k, sl]
                    xbuf[k, sl] = x
                    x2buf[k, sl] = x * x
                    acc_mx[dt, sl] = jnp.maximum(acc_mx[dt, sl], x)
                    acc_mn[dt, sl] = jnp.minimum(acc_mn[dt, sl], x)
                dbase = (dt // 16) * 16
                dwin = acc_deg[pl.ds(dbase, 16)]
                acc_deg[pl.ds(dbase, 16)] = dwin + jnp.where(
                    lane == dt - dbase, 1.0, 0.0)
                return 0

            lax.fori_loop(0, n_edges, edge_body, 0)
            pltpu.sync_copy(xbuf, ssum.at[gidx_d], add=True)
            pltpu.sync_copy(x2buf, ssq.at[gidx_d], add=True)

        def init_body(r, _):
            for j in range(8):
                sl = pl.ds(j * 16, 16)
                acc_mx[r, sl] = zeros16 - FLT_MAX
                acc_mn[r, sl] = zeros16 + FLT_MAX
            return 0
        lax.fori_loop(0, NPG, init_body, 0)

        def dinit_body(r, _):
            acc_deg[pl.ds(r * 16, 16)] = zeros16
            return 0
        lax.fori_loop(0, NPG // 16, dinit_body, 0)

        def zinit_body(r, _):
            for j in range(8):
                xbuf[r, pl.ds(j * 16, 16)] = zeros16
            return 0
        lax.fori_loop(0, FK, zinit_body, 0)
        for q in range(NPG // 64):
            sl = pl.ds(tlo + q * 64, 64)
            pltpu.sync_copy(xbuf.at[pl.ds(0, 64)], ssum.at[sl])
            pltpu.sync_copy(xbuf.at[pl.ds(0, 64)], ssq.at[sl])
        if NPG % 64:
            sl = pl.ds(tlo + (NPG // 64) * 64, NPG % 64)
            pltpu.sync_copy(xbuf.at[pl.ds(0, NPG % 64)], ssum.at[sl])
            pltpu.sync_copy(xbuf.at[pl.ds(0, NPG % 64)], ssq.at[sl])

        @pl.when(s == 0)
        def _():
            dsl = pl.ds(NPSC, 16)
            pltpu.sync_copy(xbuf.at[pl.ds(0, 16)], ssum.at[dsl])
            pltpu.sync_copy(xbuf.at[pl.ds(0, 16)], ssq.at[dsl])

        for half in range(0, FK + 16, 16):
            pend_d[pl.ds(half, 16)] = izeros16
            pend_s[pl.ds(half, 16)] = izeros16
            pend_e[pl.ds(half, 16)] = izeros16

        plsc.subcore_barrier()

        def scan_chunk(off, dbuf, sbuf, cnt):
            def filt_body(i, cnt):
                dv = dbuf[pl.ds(i * 16, 16)]
                m = (dv >= glo) & (dv < glo + NPG)
                nm = plsc.all_reduce_population_count(m)[0]
                plsc.store_compressed(pend_d.at[pl.ds(cnt, 16)],
                                      dv - sclo, mask=m)
                plsc.store_compressed(pend_s.at[pl.ds(cnt, 16)],
                                      sbuf[pl.ds(i * 16, 16)], mask=m)
                plsc.store_compressed(pend_e.at[pl.ds(cnt, 16)],
                                      off + i * 16 + lane, mask=m)
                cnt = cnt + nm
                do_flush = cnt >= FK

                @pl.when(do_flush)
                def _():
                    process_edges(FK)
                    pend_d[pl.ds(0, 16)] = pend_d[pl.ds(FK, 16)]
                    pend_s[pl.ds(0, 16)] = pend_s[pl.ds(FK, 16)]
                    pend_e[pl.ds(0, 16)] = pend_e[pl.ds(FK, 16)]

                return jnp.where(do_flush, cnt - FK, cnt)

            return lax.fori_loop(0, NITER, filt_body, cnt)

        def start_load(ci, dbuf, sbuf, semd, sems):
            off = ci * CHUNK
            pltpu.async_copy(dst_hbm.at[pl.ds(off, CHUNK)], dbuf, semd)
            pltpu.async_copy(src_hbm.at[pl.ds(off, CHUNK)], sbuf, sems)

        def wait_load(dbuf, sbuf, semd, sems):
            pltpu.make_async_copy(dst_hbm.at[pl.ds(0, CHUNK)], dbuf,
                                  semd).wait()
            pltpu.make_async_copy(src_hbm.at[pl.ds(0, CHUNK)], sbuf,
                                  sems).wait()

        start_load(0, dbuf0, sbuf0, semd0, sems0)

        def pair_body(q, cnt):
            start_load(2 * q + 1, dbuf1, sbuf1, semd1, sems1)
            wait_load(dbuf0, sbuf0, semd0, sems0)
            cnt = scan_chunk(2 * q * CHUNK, dbuf0, sbuf0, cnt)

            @pl.when(q < NCHUNK // 2 - 1)
            def _():
                start_load(2 * q + 2, dbuf0, sbuf0, semd0, sems0)

            wait_load(dbuf1, sbuf1, semd1, sems1)
            return scan_chunk((2 * q + 1) * CHUNK, dbuf1, sbuf1, cnt)

        cnt = lax.fori_loop(0, NCHUNK // 2, pair_body, jnp.int32(0))

        @pl.when(cnt > 0)
        def _():
            process_edges(cnt)

        plsc.subcore_barrier()

        pltpu.sync_copy(ssum.at[pl.ds(tlo, NPG)], o_sum.at[pl.ds(glo, NPG)])
        pltpu.sync_copy(ssq.at[pl.ds(tlo, NPG)], o_sq.at[pl.ds(glo, NPG)])
        pltpu.sync_copy(acc_mx, o_mx.at[pl.ds(glo, NPG)])
        pltpu.sync_copy(acc_mn, o_mn.at[pl.ds(glo, NPG)])
        pltpu.sync_copy(acc_deg, o_deg.at[pl.ds(glo, NPG)])

    for p in range(NPASS):
        g = p * NW + c * NS + s
        run_pass(g * NPG, (p * NW + c * NS) * NPG)
        if p + 1 < NPASS:
            plsc.subcore_barrier()


def _sc_aggregate(A, C, src, dst):
    f32 = jnp.float32
    i32 = jnp.int32
    out_type = (jax.ShapeDtypeStruct((NPAD, D), f32),
                jax.ShapeDtypeStruct((NPAD, D), f32),
                jax.ShapeDtypeStruct((NPAD, D), f32),
                jax.ShapeDtypeStruct((NPAD, D), f32),
                jax.ShapeDtypeStruct((NPAD,), f32))
    scratch = [
        pltpu.VMEM_SHARED((NPSC + 16, D), f32),
        pltpu.VMEM_SHARED((NPSC + 16, D), f32),
        pltpu.VMEM((NPG, D), f32),
        pltpu.VMEM((NPG, D), f32),
        pltpu.VMEM((NPG,), f32),
        pltpu.VMEM((CHUNK,), i32),
        pltpu.VMEM((CHUNK,), i32),
        pltpu.VMEM((CHUNK,), i32),
        pltpu.VMEM((CHUNK,), i32),
        pltpu.VMEM((FK + 16,), i32),
        pltpu.VMEM((FK + 16,), i32),
        pltpu.VMEM((FK + 16,), i32),
        pltpu.VMEM((FK,), i32),
        pltpu.VMEM((FK,), i32),
        pltpu.VMEM((FK,), i32),
        pltpu.VMEM((FK, D), f32),
        pltpu.VMEM((FK, D), f32),
        pltpu.VMEM((FK, D), f32),
        pltpu.VMEM((FK, D), f32),
        pltpu.SemaphoreType.DMA,
        pltpu.SemaphoreType.DMA,
        pltpu.SemaphoreType.DMA,
        pltpu.SemaphoreType.DMA,
        pltpu.SemaphoreType.DMA,
        pltpu.SemaphoreType.DMA,
    ]
    mesh = plsc.VectorSubcoreMesh(core_axis_name="c", subcore_axis_name="s",
                                  num_cores=NC, num_subcores=NS)
    fn = pl.kernel(_sc_body, out_type=out_type, mesh=mesh,
                   scratch_types=scratch,
                   compiler_params=pltpu.CompilerParams(
                       needs_layout_passes=False))
    return fn(A, C, dst, src)


def _post_body(h_ref, s1_ref, s2_ref, mx_ref, mn_ref, deg_ref, sn_ref,
               wpd_ref, wh_ref, wa_ref, wb_ref, wc_ref, bp_ref, o_ref):
    f32 = jnp.float32
    h = h_ref[...]
    B = jnp.dot(h, wpd_ref[...], preferred_element_type=f32)
    deg = deg_ref[...]
    s1 = s1_ref[...]
    s2 = s2_ref[...]
    degc = jnp.maximum(deg, 1.0)
    has = deg > 0.0
    mean = (s1 + deg * B) / degc
    mx = jnp.where(has, mx_ref[...] + B, 0.0)
    mn = jnp.where(has, mn_ref[...] + B, 0.0)
    mean_sq = (s2 + 2.0 * B * s1 + deg * B * B) / degc
    var = jnp.maximum(mean_sq - mean * mean, 0.0)
    std = jnp.sqrt(var + EPS)
    agg = jnp.concatenate([mean, mx, mn, std], axis=1)
    logd = jnp.log(degc + 1.0)
    amp = logd * (1.0 / AVG_D_LOG)
    att = AVG_D_LOG / logd
    acc = (jnp.dot(h, wh_ref[...], preferred_element_type=f32)
           + jnp.dot(agg, wa_ref[...], preferred_element_type=f32)
           + jnp.dot(agg * amp, wb_ref[...], preferred_element_type=f32)
           + jnp.dot(agg * att, wc_ref[...], preferred_element_type=f32))
    o_ref[...] = (acc + bp_ref[...]) * sn_ref[...]


def _tc_post(h, s1, s2, mx, mn, deg, snorm, Wpd, Wh, Wa, Wb, Wc, b_post):
    R = 2000
    node_spec = pl.BlockSpec((R, D), lambda i: (i, 0))
    col_spec = pl.BlockSpec((R, 1), lambda i: (i, 0))
    full = lambda r, c: pl.BlockSpec((r, c), lambda i: (0, 0))
    return pl.pallas_call(
        _post_body,
        grid=(5,),
        in_specs=[node_spec, node_spec, node_spec, node_spec, node_spec,
                  col_spec, col_spec,
                  full(D, D), full(D, D), full(512, D), full(512, D),
                  full(512, D), full(1, D)],
        out_specs=node_spec,
        out_shape=jax.ShapeDtypeStruct((N, D), jnp.float32),
    )(h, s1, s2, mx, mn, deg, snorm, Wpd, Wh, Wa, Wb, Wc, b_post)


def kernel(h, edge_index, e, snorm_n, W_pre, b_pre, W_post, b_post):
    Wps = W_pre[0:D]
    Wpd = W_pre[D:2 * D]
    Wpe = W_pre[2 * D:]
    Wh = W_post[0:D]
    Wa = W_post[D:D + 512]
    Wb = W_post[D + 512:D + 1024]
    Wc = W_post[D + 1024:]
    src = edge_index[0]
    dst = edge_index[1]

    A, C = _tc_pre(h, e, Wps, Wpe, b_pre)
    s1, s2, mx, mn, deg = _sc_aggregate(A, C, src, dst)
    out = _tc_post(h, s1[:N], s2[:N], mx[:N], mn[:N],
                   deg[:N].reshape(N, 1), snorm_n,
                   Wpd, Wh, Wa, Wb, Wc, b_post.reshape(1, D))
    return out


# speedup vs baseline: 3.1328x; 1.5114x over previous
import jax
import jax.numpy as jnp
from jax import lax
from jax.experimental import pallas as pl
from jax.experimental.pallas import tpu as pltpu
from jax.experimental.pallas import tpu_sc as plsc

N = 10000
E = 320000
D = 128
D_EDGE = 16
AVG_D_LOG = 3.4965075614664802
EPS = 1e-5

NC = 2
NS = 16
NW = NC * NS
NPASS = 2
NG = NW * NPASS
NPG = 160
NPAD = NG * NPG
NPSC = NS * NPG
DUMMY = NPSC
CHUNK = 2000
NITER = CHUNK // 16
NCHUNK = E // CHUNK
FLT_MAX = 3.4028235e38
FK = 64


def _pre_node_body(h_ref, w_ref, o_ref):
    o_ref[...] = jnp.dot(h_ref[...], w_ref[...],
                         preferred_element_type=jnp.float32)


def _pre_edge_body(e_ref, w_ref, b_ref, o_ref):
    o_ref[...] = jnp.dot(e_ref[...], w_ref[...],
                         preferred_element_type=jnp.float32) + b_ref[...]


def _tc_pre(h, e, Wps, Wpe, b_pre):
    A = pl.pallas_call(
        _pre_node_body,
        grid=(5,),
        in_specs=[pl.BlockSpec((2000, D), lambda i: (i, 0)),
                  pl.BlockSpec((D, D), lambda i: (0, 0))],
        out_specs=pl.BlockSpec((2000, D), lambda i: (i, 0)),
        out_shape=jax.ShapeDtypeStruct((N, D), jnp.float32),
    )(h, Wps)
    C = pl.pallas_call(
        _pre_edge_body,
        grid=(160,),
        in_specs=[pl.BlockSpec((CHUNK, D_EDGE), lambda i: (i, 0)),
                  pl.BlockSpec((D_EDGE, D), lambda i: (0, 0)),
                  pl.BlockSpec((1, D), lambda i: (0, 0))],
        out_specs=pl.BlockSpec((CHUNK, D), lambda i: (i, 0)),
        out_shape=jax.ShapeDtypeStruct((E, D), jnp.float32),
    )(e, Wpe, b_pre.reshape(1, D))
    return A, C


def _sc_body(A_hbm, C_hbm, dst_hbm, src_hbm,
             o_sum, o_sq, o_mx, o_mn, o_deg,
             ssum, ssq,
             acc_mx, acc_mn, acc_deg,
             dbuf0, sbuf0, dbuf1, sbuf1,
             pend_d, pend_s, pend_e,
             gidx_d, gidx_s, gidx_e,
             astage, cstage, xbuf, x2buf,
             sem_a, sem_c, semd0, sems0, semd1, sems1):
    c = lax.axis_index("c")
    s = lax.axis_index("s")
    lane = lax.iota(jnp.int32, 16)
    zeros16 = jnp.zeros((16,), jnp.float32)
    izeros16 = jnp.zeros((16,), jnp.int32)
    tlo = s * NPG

    def run_pass(glo, sclo):

        def process_edges(n_edges):
            for q in range(FK // 16):
                sl16 = pl.ds(q * 16, 16)
                gidx_s[sl16] = pend_s[sl16]
                gidx_e[sl16] = pend_e[sl16]
                active = (q * 16 + lane) < n_edges
                gidx_d[sl16] = jnp.where(active, pend_d[sl16], DUMMY)
            cp_a = pltpu.async_copy(A_hbm.at[gidx_s], astage, sem_a)
            cp_c = pltpu.async_copy(C_hbm.at[gidx_e], cstage, sem_c)
            cp_a.wait()
            cp_c.wait()

            def edge_body(k, _):
                gbase = (k // 16) * 16
                dvec = pend_d[pl.ds(gbase, 16)]
                dnums = lax.GatherDimensionNumbers(
                    offset_dims=(), collapsed_slice_dims=(0,),
                    start_index_map=(0,))
                dl = lax.gather(dvec, jnp.full((16, 1), 0, jnp.int32)
                                + (k - gbase), dnums, (1,),
                                mode=lax.GatherScatterMode.PROMISE_IN_BOUNDS,
                                )[0]
                dt = dl - tlo
                sls = [pl.ds(j * 16, 16) for j in range(8)]
                xs = [astage[k, sl] + cstage[k, sl] for sl in sls]
                for j, sl in enumerate(sls):
                    xbuf[k, sl] = xs[j]
                for j, sl in enumerate(sls):
                    x2buf[k, sl] = xs[j] * xs[j]
                mxv = [acc_mx[dt, sl] for sl in sls]
                mnv = [acc_mn[dt, sl] for sl in sls]
                for j, sl in enumerate(sls):
                    acc_mx[dt, sl] = jnp.maximum(mxv[j], xs[j])
                for j, sl in enumerate(sls):
                    acc_mn[dt, sl] = jnp.minimum(mnv[j], xs[j])
                dbase = (dt // 16) * 16
                dwin = acc_deg[pl.ds(dbase, 16)]
                acc_deg[pl.ds(dbase, 16)] = dwin + jnp.where(
                    lane == dt - dbase, 1.0, 0.0)
                return 0

            lax.fori_loop(0, n_edges, edge_body, 0)
            pltpu.sync_copy(xbuf, ssum.at[gidx_d], add=True)
            pltpu.sync_copy(x2buf, ssq.at[gidx_d], add=True)

        def init_body(r, _):
            for j in range(8):
                sl = pl.ds(j * 16, 16)
                acc_mx[r, sl] = zeros16 - FLT_MAX
                acc_mn[r, sl] = zeros16 + FLT_MAX
            return 0
        lax.fori_loop(0, NPG, init_body, 0)

        def dinit_body(r, _):
            acc_deg[pl.ds(r * 16, 16)] = zeros16
            return 0
        lax.fori_loop(0, NPG // 16, dinit_body, 0)

        def zinit_body(r, _):
            for j in range(8):
                xbuf[r, pl.ds(j * 16, 16)] = zeros16
            return 0
        lax.fori_loop(0, FK, zinit_body, 0)
        for q in range(NPG // 64):
            sl = pl.ds(tlo + q * 64, 64)
            pltpu.sync_copy(xbuf.at[pl.ds(0, 64)], ssum.at[sl])
            pltpu.sync_copy(xbuf.at[pl.ds(0, 64)], ssq.at[sl])
        if NPG % 64:
            sl = pl.ds(tlo + (NPG // 64) * 64, NPG % 64)
            pltpu.sync_copy(xbuf.at[pl.ds(0, NPG % 64)], ssum.at[sl])
            pltpu.sync_copy(xbuf.at[pl.ds(0, NPG % 64)], ssq.at[sl])

        @pl.when(s == 0)
        def _():
            dsl = pl.ds(NPSC, 16)
            pltpu.sync_copy(xbuf.at[pl.ds(0, 16)], ssum.at[dsl])
            pltpu.sync_copy(xbuf.at[pl.ds(0, 16)], ssq.at[dsl])

        for half in range(0, FK + 32, 16):
            pend_d[pl.ds(half, 16)] = izeros16
            pend_s[pl.ds(half, 16)] = izeros16
            pend_e[pl.ds(half, 16)] = izeros16

        plsc.subcore_barrier()

        def scan_chunk(off, dbuf, sbuf, cnt):
            def filt_body(i, cnt):
                for u in range(2):
                    sl16 = pl.ds(i * 32 + u * 16, 16)
                    dv = dbuf[sl16]
                    m = (dv >= glo) & (dv < glo + NPG)
                    nm = plsc.all_reduce_population_count(m)[0]
                    plsc.store_compressed(pend_d.at[pl.ds(cnt, 16)],
                                          dv - sclo, mask=m)
                    plsc.store_compressed(pend_s.at[pl.ds(cnt, 16)],
                                          sbuf[sl16], mask=m)
                    plsc.store_compressed(pend_e.at[pl.ds(cnt, 16)],
                                          off + i * 32 + u * 16 + lane,
                                          mask=m)
                    cnt = cnt + nm
                do_flush = cnt >= FK

                @pl.when(do_flush)
                def _():
                    process_edges(FK)
                    for half in (0, 16):
                        mv = pl.ds(half, 16)
                        mv2 = pl.ds(FK + half, 16)
                        pend_d[mv] = pend_d[mv2]
                        pend_s[mv] = pend_s[mv2]
                        pend_e[mv] = pend_e[mv2]

                return jnp.where(do_flush, cnt - FK, cnt)

            return lax.fori_loop(0, NITER // 2, filt_body, cnt)

        def start_load(ci, dbuf, sbuf, semd, sems):
            off = ci * CHUNK
            pltpu.async_copy(dst_hbm.at[pl.ds(off, CHUNK)], dbuf, semd)
            pltpu.async_copy(src_hbm.at[pl.ds(off, CHUNK)], sbuf, sems)

        def wait_load(dbuf, sbuf, semd, sems):
            pltpu.make_async_copy(dst_hbm.at[pl.ds(0, CHUNK)], dbuf,
                                  semd).wait()
            pltpu.make_async_copy(src_hbm.at[pl.ds(0, CHUNK)], sbuf,
                                  sems).wait()

        start_load(0, dbuf0, sbuf0, semd0, sems0)

        def pair_body(q, cnt):
            start_load(2 * q + 1, dbuf1, sbuf1, semd1, sems1)
            wait_load(dbuf0, sbuf0, semd0, sems0)
            cnt = scan_chunk(2 * q * CHUNK, dbuf0, sbuf0, cnt)

            @pl.when(q < NCHUNK // 2 - 1)
            def _():
                start_load(2 * q + 2, dbuf0, sbuf0, semd0, sems0)

            wait_load(dbuf1, sbuf1, semd1, sems1)
            return scan_chunk((2 * q + 1) * CHUNK, dbuf1, sbuf1, cnt)

        cnt = lax.fori_loop(0, NCHUNK // 2, pair_body, jnp.int32(0))

        @pl.when(cnt > 0)
        def _():
            process_edges(cnt)

        plsc.subcore_barrier()

        pltpu.sync_copy(ssum.at[pl.ds(tlo, NPG)], o_sum.at[pl.ds(glo, NPG)])
        pltpu.sync_copy(ssq.at[pl.ds(tlo, NPG)], o_sq.at[pl.ds(glo, NPG)])
        pltpu.sync_copy(acc_mx, o_mx.at[pl.ds(glo, NPG)])
        pltpu.sync_copy(acc_mn, o_mn.at[pl.ds(glo, NPG)])
        pltpu.sync_copy(acc_deg, o_deg.at[pl.ds(glo, NPG)])

    for p in range(NPASS):
        g = p * NW + c * NS + s
        run_pass(g * NPG, (p * NW + c * NS) * NPG)
        if p + 1 < NPASS:
            plsc.subcore_barrier()


def _sc_aggregate(A, C, src, dst):
    f32 = jnp.float32
    i32 = jnp.int32
    out_type = (jax.ShapeDtypeStruct((NPAD, D), f32),
                jax.ShapeDtypeStruct((NPAD, D), f32),
                jax.ShapeDtypeStruct((NPAD, D), f32),
                jax.ShapeDtypeStruct((NPAD, D), f32),
                jax.ShapeDtypeStruct((NPAD,), f32))
    scratch = [
        pltpu.VMEM_SHARED((NPSC + 16, D), f32),
        pltpu.VMEM_SHARED((NPSC + 16, D), f32),
        pltpu.VMEM((NPG, D), f32),
        pltpu.VMEM((NPG, D), f32),
        pltpu.VMEM((NPG,), f32),
        pltpu.VMEM((CHUNK,), i32),
        pltpu.VMEM((CHUNK,), i32),
        pltpu.VMEM((CHUNK,), i32),
        pltpu.VMEM((CHUNK,), i32),
        pltpu.VMEM((FK + 32,), i32),
        pltpu.VMEM((FK + 32,), i32),
        pltpu.VMEM((FK + 32,), i32),
        pltpu.VMEM((FK,), i32),
        pltpu.VMEM((FK,), i32),
        pltpu.VMEM((FK,), i32),
        pltpu.VMEM((FK, D), f32),
        pltpu.VMEM((FK, D), f32),
        pltpu.VMEM((FK, D), f32),
        pltpu.VMEM((FK, D), f32),
        pltpu.SemaphoreType.DMA,
        pltpu.SemaphoreType.DMA,
        pltpu.SemaphoreType.DMA,
        pltpu.SemaphoreType.DMA,
        pltpu.SemaphoreType.DMA,
        pltpu.SemaphoreType.DMA,
    ]
    mesh = plsc.VectorSubcoreMesh(core_axis_name="c", subcore_axis_name="s",
                                  num_cores=NC, num_subcores=NS)
    fn = pl.kernel(_sc_body, out_type=out_type, mesh=mesh,
                   scratch_types=scratch,
                   compiler_params=pltpu.CompilerParams(
                       needs_layout_passes=False))
    return fn(A, C, dst, src)


def _post_body(h_ref, s1_ref, s2_ref, mx_ref, mn_ref, deg_ref, sn_ref,
               wpd_ref, wh_ref, wa_ref, wb_ref, wc_ref, bp_ref, o_ref):
    f32 = jnp.float32
    h = h_ref[...]
    B = jnp.dot(h, wpd_ref[...], preferred_element_type=f32)
    deg = deg_ref[...]
    s1 = s1_ref[...]
    s2 = s2_ref[...]
    degc = jnp.maximum(deg, 1.0)
    has = deg > 0.0
    mean = (s1 + deg * B) / degc
    mx = jnp.where(has, mx_ref[...] + B, 0.0)
    mn = jnp.where(has, mn_ref[...] + B, 0.0)
    mean_sq = (s2 + 2.0 * B * s1 + deg * B * B) / degc
    var = jnp.maximum(mean_sq - mean * mean, 0.0)
    std = jnp.sqrt(var + EPS)
    agg = jnp.concatenate([mean, mx, mn, std], axis=1)
    logd = jnp.log(degc + 1.0)
    amp = logd * (1.0 / AVG_D_LOG)
    att = AVG_D_LOG / logd
    acc = (jnp.dot(h, wh_ref[...], preferred_element_type=f32)
           + jnp.dot(agg, wa_ref[...], preferred_element_type=f32)
           + jnp.dot(agg * amp, wb_ref[...], preferred_element_type=f32)
           + jnp.dot(agg * att, wc_ref[...], preferred_element_type=f32))
    o_ref[...] = (acc + bp_ref[...]) * sn_ref[...]


def _tc_post(h, s1, s2, mx, mn, deg, snorm, Wpd, Wh, Wa, Wb, Wc, b_post):
    R = 2000
    node_spec = pl.BlockSpec((R, D), lambda i: (i, 0))
    col_spec = pl.BlockSpec((R, 1), lambda i: (i, 0))
    full = lambda r, c: pl.BlockSpec((r, c), lambda i: (0, 0))
    return pl.pallas_call(
        _post_body,
        grid=(5,),
        in_specs=[node_spec, node_spec, node_spec, node_spec, node_spec,
                  col_spec, col_spec,
                  full(D, D), full(D, D), full(512, D), full(512, D),
                  full(512, D), full(1, D)],
        out_specs=node_spec,
        out_shape=jax.ShapeDtypeStruct((N, D), jnp.float32),
    )(h, s1, s2, mx, mn, deg, snorm, Wpd, Wh, Wa, Wb, Wc, b_post)


def kernel(h, edge_index, e, snorm_n, W_pre, b_pre, W_post, b_post):
    Wps = W_pre[0:D]
    Wpd = W_pre[D:2 * D]
    Wpe = W_pre[2 * D:]
    Wh = W_post[0:D]
    Wa = W_post[D:D + 512]
    Wb = W_post[D + 512:D + 1024]
    Wc = W_post[D + 1024:]
    src = edge_index[0]
    dst = edge_index[1]

    A, C = _tc_pre(h, e, Wps, Wpe, b_pre)
    s1, s2, mx, mn, deg = _sc_aggregate(A, C, src, dst)
    out = _tc_post(h, s1[:N], s2[:N], mx[:N], mn[:N],
                   deg[:N].reshape(N, 1), snorm_n,
                   Wpd, Wh, Wa, Wb, Wc, b_post.reshape(1, D))
    return out


# speedup vs baseline: 3.1813x; 1.0155x over previous
import jax
import jax.numpy as jnp
from jax import lax
from jax.experimental import pallas as pl
from jax.experimental.pallas import tpu as pltpu
from jax.experimental.pallas import tpu_sc as plsc

N = 10000
E = 320000
D = 128
D_EDGE = 16
AVG_D_LOG = 3.4965075614664802
EPS = 1e-5

NC = 2
NS = 16
NW = NC * NS
NPASS = 2
NG = NW * NPASS
NPG = 160
NPAD = NG * NPG
NPSC = NS * NPG
DUMMY = NPSC
CHUNK = 3200
NITER = CHUNK // 16
NCHUNK = E // CHUNK
FLT_MAX = 3.4028235e38
FK = 64


def _pre_node_body(h_ref, w_ref, o_ref):
    o_ref[...] = jnp.dot(h_ref[...], w_ref[...],
                         preferred_element_type=jnp.float32)


def _pre_edge_body(e_ref, w_ref, b_ref, o_ref):
    o_ref[...] = jnp.dot(e_ref[...], w_ref[...],
                         preferred_element_type=jnp.float32) + b_ref[...]


def _tc_pre(h, e, Wps, Wpe, b_pre):
    A = pl.pallas_call(
        _pre_node_body,
        grid=(5,),
        in_specs=[pl.BlockSpec((2000, D), lambda i: (i, 0)),
                  pl.BlockSpec((D, D), lambda i: (0, 0))],
        out_specs=pl.BlockSpec((2000, D), lambda i: (i, 0)),
        out_shape=jax.ShapeDtypeStruct((N, D), jnp.float32),
    )(h, Wps)
    C = pl.pallas_call(
        _pre_edge_body,
        grid=(E // CHUNK,),
        in_specs=[pl.BlockSpec((CHUNK, D_EDGE), lambda i: (i, 0)),
                  pl.BlockSpec((D_EDGE, D), lambda i: (0, 0)),
                  pl.BlockSpec((1, D), lambda i: (0, 0))],
        out_specs=pl.BlockSpec((CHUNK, D), lambda i: (i, 0)),
        out_shape=jax.ShapeDtypeStruct((E, D), jnp.float32),
    )(e, Wpe, b_pre.reshape(1, D))
    return A, C


def _sc_body(A_hbm, C_hbm, dst_hbm, src_hbm,
             o_sum, o_sq, o_mx, o_mn, o_deg,
             ssum, ssq,
             acc_mx, acc_mn, acc_deg,
             dbuf0, sbuf0, dbuf1, sbuf1,
             pend_d, pend_s, pend_e,
             gidx_d, gidx_s, gidx_e,
             astage, cstage, xbuf, x2buf,
             sem_a, sem_c, semd0, sems0, semd1, sems1):
    c = lax.axis_index("c")
    s = lax.axis_index("s")
    lane = lax.iota(jnp.int32, 16)
    zeros16 = jnp.zeros((16,), jnp.float32)
    izeros16 = jnp.zeros((16,), jnp.int32)
    tlo = s * NPG

    def run_pass(glo, sclo):

        def process_edges(n_edges):
            for q in range(FK // 16):
                sl16 = pl.ds(q * 16, 16)
                gidx_s[sl16] = pend_s[sl16]
                gidx_e[sl16] = pend_e[sl16]
                active = (q * 16 + lane) < n_edges
                gidx_d[sl16] = jnp.where(active, pend_d[sl16], DUMMY)
            cp_a = pltpu.async_copy(A_hbm.at[gidx_s], astage, sem_a)
            cp_c = pltpu.async_copy(C_hbm.at[gidx_e], cstage, sem_c)
            cp_a.wait()
            cp_c.wait()

            def edge_body(k, _):
                gbase = (k // 16) * 16
                dvec = pend_d[pl.ds(gbase, 16)]
                dnums = lax.GatherDimensionNumbers(
                    offset_dims=(), collapsed_slice_dims=(0,),
                    start_index_map=(0,))
                dl = lax.gather(dvec, jnp.full((16, 1), 0, jnp.int32)
                                + (k - gbase), dnums, (1,),
                                mode=lax.GatherScatterMode.PROMISE_IN_BOUNDS,
                                )[0]
                dt = dl - tlo
                sls = [pl.ds(j * 16, 16) for j in range(8)]
                xs = [astage[k, sl] + cstage[k, sl] for sl in sls]
                for j, sl in enumerate(sls):
                    xbuf[k, sl] = xs[j]
                for j, sl in enumerate(sls):
                    x2buf[k, sl] = xs[j] * xs[j]
                mxv = [acc_mx[dt, sl] for sl in sls]
                mnv = [acc_mn[dt, sl] for sl in sls]
                for j, sl in enumerate(sls):
                    acc_mx[dt, sl] = jnp.maximum(mxv[j], xs[j])
                for j, sl in enumerate(sls):
                    acc_mn[dt, sl] = jnp.minimum(mnv[j], xs[j])
                dbase = (dt // 16) * 16
                dwin = acc_deg[pl.ds(dbase, 16)]
                acc_deg[pl.ds(dbase, 16)] = dwin + jnp.where(
                    lane == dt - dbase, 1.0, 0.0)
                return 0

            lax.fori_loop(0, n_edges, edge_body, 0)
            pltpu.sync_copy(xbuf, ssum.at[gidx_d], add=True)
            pltpu.sync_copy(x2buf, ssq.at[gidx_d], add=True)

        def init_body(r, _):
            for j in range(8):
                sl = pl.ds(j * 16, 16)
                acc_mx[r, sl] = zeros16 - FLT_MAX
                acc_mn[r, sl] = zeros16 + FLT_MAX
            return 0
        lax.fori_loop(0, NPG, init_body, 0)

        def dinit_body(r, _):
            acc_deg[pl.ds(r * 16, 16)] = zeros16
            return 0
        lax.fori_loop(0, NPG // 16, dinit_body, 0)

        def zinit_body(r, _):
            for j in range(8):
                xbuf[r, pl.ds(j * 16, 16)] = zeros16
            return 0
        lax.fori_loop(0, FK, zinit_body, 0)
        for q in range(NPG // 64):
            sl = pl.ds(tlo + q * 64, 64)
            pltpu.sync_copy(xbuf.at[pl.ds(0, 64)], ssum.at[sl])
            pltpu.sync_copy(xbuf.at[pl.ds(0, 64)], ssq.at[sl])
        if NPG % 64:
            sl = pl.ds(tlo + (NPG // 64) * 64, NPG % 64)
            pltpu.sync_copy(xbuf.at[pl.ds(0, NPG % 64)], ssum.at[sl])
            pltpu.sync_copy(xbuf.at[pl.ds(0, NPG % 64)], ssq.at[sl])

        @pl.when(s == 0)
        def _():
            dsl = pl.ds(NPSC, 16)
            pltpu.sync_copy(xbuf.at[pl.ds(0, 16)], ssum.at[dsl])
            pltpu.sync_copy(xbuf.at[pl.ds(0, 16)], ssq.at[dsl])

        for half in range(0, FK + 32, 16):
            pend_d[pl.ds(half, 16)] = izeros16
            pend_s[pl.ds(half, 16)] = izeros16
            pend_e[pl.ds(half, 16)] = izeros16

        plsc.subcore_barrier()

        def scan_chunk(off, dbuf, sbuf, cnt):
            def filt_body(i, cnt):
                for u in range(2):
                    sl16 = pl.ds(i * 32 + u * 16, 16)
                    dv = dbuf[sl16]
                    m = (dv >= glo) & (dv < glo + NPG)
                    nm = plsc.all_reduce_population_count(m)[0]
                    plsc.store_compressed(pend_d.at[pl.ds(cnt, 16)],
                                          dv - sclo, mask=m)
                    plsc.store_compressed(pend_s.at[pl.ds(cnt, 16)],
                                          sbuf[sl16], mask=m)
                    plsc.store_compressed(pend_e.at[pl.ds(cnt, 16)],
                                          off + i * 32 + u * 16 + lane,
                                          mask=m)
                    cnt = cnt + nm
                do_flush = cnt >= FK

                @pl.when(do_flush)
                def _():
                    process_edges(FK)
                    for half in (0, 16):
                        mv = pl.ds(half, 16)
                        mv2 = pl.ds(FK + half, 16)
                        pend_d[mv] = pend_d[mv2]
                        pend_s[mv] = pend_s[mv2]
                        pend_e[mv] = pend_e[mv2]

                return jnp.where(do_flush, cnt - FK, cnt)

            return lax.fori_loop(0, NITER // 2, filt_body, cnt)

        def start_load(ci, dbuf, sbuf, semd, sems):
            off = ci * CHUNK
            pltpu.async_copy(dst_hbm.at[pl.ds(off, CHUNK)], dbuf, semd)
            pltpu.async_copy(src_hbm.at[pl.ds(off, CHUNK)], sbuf, sems)

        def wait_load(dbuf, sbuf, semd, sems):
            pltpu.make_async_copy(dst_hbm.at[pl.ds(0, CHUNK)], dbuf,
                                  semd).wait()
            pltpu.make_async_copy(src_hbm.at[pl.ds(0, CHUNK)], sbuf,
                                  sems).wait()

        start_load(0, dbuf0, sbuf0, semd0, sems0)

        def pair_body(q, cnt):
            start_load(2 * q + 1, dbuf1, sbuf1, semd1, sems1)
            wait_load(dbuf0, sbuf0, semd0, sems0)
            cnt = scan_chunk(2 * q * CHUNK, dbuf0, sbuf0, cnt)

            @pl.when(q < NCHUNK // 2 - 1)
            def _():
                start_load(2 * q + 2, dbuf0, sbuf0, semd0, sems0)

            wait_load(dbuf1, sbuf1, semd1, sems1)
            return scan_chunk((2 * q + 1) * CHUNK, dbuf1, sbuf1, cnt)

        cnt = lax.fori_loop(0, NCHUNK // 2, pair_body, jnp.int32(0))

        @pl.when(cnt > 0)
        def _():
            process_edges(cnt)

        plsc.subcore_barrier()

        pltpu.sync_copy(ssum.at[pl.ds(tlo, NPG)], o_sum.at[pl.ds(glo, NPG)])
        pltpu.sync_copy(ssq.at[pl.ds(tlo, NPG)], o_sq.at[pl.ds(glo, NPG)])
        pltpu.sync_copy(acc_mx, o_mx.at[pl.ds(glo, NPG)])
        pltpu.sync_copy(acc_mn, o_mn.at[pl.ds(glo, NPG)])
        pltpu.sync_copy(acc_deg, o_deg.at[pl.ds(glo, NPG)])

    for p in range(NPASS):
        g = p * NW + c * NS + s
        run_pass(g * NPG, (p * NW + c * NS) * NPG)
        if p + 1 < NPASS:
            plsc.subcore_barrier()


def _sc_aggregate(A, C, src, dst):
    f32 = jnp.float32
    i32 = jnp.int32
    out_type = (jax.ShapeDtypeStruct((NPAD, D), f32),
                jax.ShapeDtypeStruct((NPAD, D), f32),
                jax.ShapeDtypeStruct((NPAD, D), f32),
                jax.ShapeDtypeStruct((NPAD, D), f32),
                jax.ShapeDtypeStruct((NPAD,), f32))
    scratch = [
        pltpu.VMEM_SHARED((NPSC + 16, D), f32),
        pltpu.VMEM_SHARED((NPSC + 16, D), f32),
        pltpu.VMEM((NPG, D), f32),
        pltpu.VMEM((NPG, D), f32),
        pltpu.VMEM((NPG,), f32),
        pltpu.VMEM((CHUNK,), i32),
        pltpu.VMEM((CHUNK,), i32),
        pltpu.VMEM((CHUNK,), i32),
        pltpu.VMEM((CHUNK,), i32),
        pltpu.VMEM((FK + 32,), i32),
        pltpu.VMEM((FK + 32,), i32),
        pltpu.VMEM((FK + 32,), i32),
        pltpu.VMEM((FK,), i32),
        pltpu.VMEM((FK,), i32),
        pltpu.VMEM((FK,), i32),
        pltpu.VMEM((FK, D), f32),
        pltpu.VMEM((FK, D), f32),
        pltpu.VMEM((FK, D), f32),
        pltpu.VMEM((FK, D), f32),
        pltpu.SemaphoreType.DMA,
        pltpu.SemaphoreType.DMA,
        pltpu.SemaphoreType.DMA,
        pltpu.SemaphoreType.DMA,
        pltpu.SemaphoreType.DMA,
        pltpu.SemaphoreType.DMA,
    ]
    mesh = plsc.VectorSubcoreMesh(core_axis_name="c", subcore_axis_name="s",
                                  num_cores=NC, num_subcores=NS)
    fn = pl.kernel(_sc_body, out_type=out_type, mesh=mesh,
                   scratch_types=scratch,
                   compiler_params=pltpu.CompilerParams(
                       needs_layout_passes=False))
    return fn(A, C, dst, src)


def _post_body(h_ref, s1_ref, s2_ref, mx_ref, mn_ref, deg_ref, sn_ref,
               wpd_ref, wh_ref, wa_ref, wb_ref, wc_ref, bp_ref, o_ref):
    f32 = jnp.float32
    h = h_ref[...]
    B = jnp.dot(h, wpd_ref[...], preferred_element_type=f32)
    deg = deg_ref[...]
    s1 = s1_ref[...]
    s2 = s2_ref[...]
    degc = jnp.maximum(deg, 1.0)
    has = deg > 0.0
    mean = (s1 + deg * B) / degc
    mx = jnp.where(has, mx_ref[...] + B, 0.0)
    mn = jnp.where(has, mn_ref[...] + B, 0.0)
    mean_sq = (s2 + 2.0 * B * s1 + deg * B * B) / degc
    var = jnp.maximum(mean_sq - mean * mean, 0.0)
    std = jnp.sqrt(var + EPS)
    agg = jnp.concatenate([mean, mx, mn, std], axis=1)
    logd = jnp.log(degc + 1.0)
    amp = logd * (1.0 / AVG_D_LOG)
    att = AVG_D_LOG / logd
    acc = (jnp.dot(h, wh_ref[...], preferred_element_type=f32)
           + jnp.dot(agg, wa_ref[...], preferred_element_type=f32)
           + jnp.dot(agg * amp, wb_ref[...], preferred_element_type=f32)
           + jnp.dot(agg * att, wc_ref[...], preferred_element_type=f32))
    o_ref[...] = (acc + bp_ref[...]) * sn_ref[...]


def _tc_post(h, s1, s2, mx, mn, deg, snorm, Wpd, Wh, Wa, Wb, Wc, b_post):
    R = 2000
    node_spec = pl.BlockSpec((R, D), lambda i: (i, 0))
    col_spec = pl.BlockSpec((R, 1), lambda i: (i, 0))
    full = lambda r, c: pl.BlockSpec((r, c), lambda i: (0, 0))
    return pl.pallas_call(
        _post_body,
        grid=(5,),
        in_specs=[node_spec, node_spec, node_spec, node_spec, node_spec,
                  col_spec, col_spec,
                  full(D, D), full(D, D), full(512, D), full(512, D),
                  full(512, D), full(1, D)],
        out_specs=node_spec,
        out_shape=jax.ShapeDtypeStruct((N, D), jnp.float32),
    )(h, s1, s2, mx, mn, deg, snorm, Wpd, Wh, Wa, Wb, Wc, b_post)


def kernel(h, edge_index, e, snorm_n, W_pre, b_pre, W_post, b_post):
    Wps = W_pre[0:D]
    Wpd = W_pre[D:2 * D]
    Wpe = W_pre[2 * D:]
    Wh = W_post[0:D]
    Wa = W_post[D:D + 512]
    Wb = W_post[D + 512:D + 1024]
    Wc = W_post[D + 1024:]
    src = edge_index[0]
    dst = edge_index[1]

    A, C = _tc_pre(h, e, Wps, Wpe, b_pre)
    s1, s2, mx, mn, deg = _sc_aggregate(A, C, src, dst)
    out = _tc_post(h, s1[:N], s2[:N], mx[:N], mn[:N],
                   deg[:N].reshape(N, 1), snorm_n,
                   Wpd, Wh, Wa, Wb, Wc, b_post.reshape(1, D))
    return out


# speedup vs baseline: 3.3303x; 1.0468x over previous
import jax
import jax.numpy as jnp
from jax import lax
from jax.experimental import pallas as pl
from jax.experimental.pallas import tpu as pltpu
from jax.experimental.pallas import tpu_sc as plsc

N = 10000
E = 320000
D = 128
D_EDGE = 16
AVG_D_LOG = 3.4965075614664802
EPS = 1e-5

NC = 2
NS = 16
NW = NC * NS
NPASS = 2
NG = NW * NPASS
NPG = 160
NPAD = NG * NPG
NPSC = NS * NPG
DUMMY = NPSC
CHUNK = 3200
NITER = CHUNK // 16
NCHUNK = E // CHUNK
FLT_MAX = 3.4028235e38
FK = 64


def _pre_node_body(h_ref, w_ref, o_ref):
    o_ref[...] = jnp.dot(h_ref[...], w_ref[...],
                         preferred_element_type=jnp.float32)


def _pre_edge_body(e_ref, w_ref, b_ref, o_ref):
    o_ref[...] = jnp.dot(e_ref[...], w_ref[...],
                         preferred_element_type=jnp.float32) + b_ref[...]


def _tc_pre(h, e, Wps, Wpe, b_pre):
    A = pl.pallas_call(
        _pre_node_body,
        grid=(5,),
        in_specs=[pl.BlockSpec((2000, D), lambda i: (i, 0)),
                  pl.BlockSpec((D, D), lambda i: (0, 0))],
        out_specs=pl.BlockSpec((2000, D), lambda i: (i, 0)),
        out_shape=jax.ShapeDtypeStruct((N, D), jnp.float32),
    )(h, Wps)
    C = pl.pallas_call(
        _pre_edge_body,
        grid=(E // CHUNK,),
        in_specs=[pl.BlockSpec((CHUNK, D_EDGE), lambda i: (i, 0)),
                  pl.BlockSpec((D_EDGE, D), lambda i: (0, 0)),
                  pl.BlockSpec((1, D), lambda i: (0, 0))],
        out_specs=pl.BlockSpec((CHUNK, D), lambda i: (i, 0)),
        out_shape=jax.ShapeDtypeStruct((E, D), jnp.float32),
    )(e, Wpe, b_pre.reshape(1, D))
    return A, C


def _sc_body(A_hbm, C_hbm, dst_hbm, src_hbm,
             o_sum, o_sq, o_mx, o_mn, o_deg,
             ssum, ssq,
             acc_mx, acc_mn, acc_deg,
             dbuf0, sbuf0, dbuf1, sbuf1,
             pend_d, pend_s, pend_e,
             gidx_d, gidx_s, gidx_e,
             astage, cstage, xbuf, x2buf,
             sem_a, sem_c, semd0, sems0, semd1, sems1):
    c = lax.axis_index("c")
    s = lax.axis_index("s")
    lane = lax.iota(jnp.int32, 16)
    zeros16 = jnp.zeros((16,), jnp.float32)
    izeros16 = jnp.zeros((16,), jnp.int32)
    tlo = s * NPG
    def run_pass(glo, sclo):

        def process_edges(n_edges):
            for q in range(FK // 16):
                sl16 = pl.ds(q * 16, 16)
                gidx_s[sl16] = pend_s[sl16]
                gidx_e[sl16] = pend_e[sl16]
                active = (q * 16 + lane) < n_edges
                gidx_d[sl16] = jnp.where(active, pend_d[sl16], DUMMY)
            cp_a = pltpu.async_copy(A_hbm.at[gidx_s], astage, sem_a)
            cp_c = pltpu.async_copy(C_hbm.at[gidx_e], cstage, sem_c)
            cp_a.wait()
            cp_c.wait()

            def edge_body(k, _):
                gbase = (k // 16) * 16
                dvec = pend_d[pl.ds(gbase, 16)]
                dnums = lax.GatherDimensionNumbers(
                    offset_dims=(), collapsed_slice_dims=(0,),
                    start_index_map=(0,))
                dl = lax.gather(dvec, jnp.full((16, 1), 0, jnp.int32)
                                + (k - gbase), dnums, (1,),
                                mode=lax.GatherScatterMode.PROMISE_IN_BOUNDS,
                                )[0]
                dt = dl - tlo
                sls = [pl.ds(j * 16, 16) for j in range(8)]
                xs = [astage[k, sl] + cstage[k, sl] for sl in sls]
                for j, sl in enumerate(sls):
                    xbuf[k, sl] = xs[j]
                for j, sl in enumerate(sls):
                    x2buf[k, sl] = xs[j] * xs[j]
                mxv = [acc_mx[dt, sl] for sl in sls]
                mnv = [acc_mn[dt, sl] for sl in sls]
                for j, sl in enumerate(sls):
                    acc_mx[dt, sl] = jnp.maximum(mxv[j], xs[j])
                for j, sl in enumerate(sls):
                    acc_mn[dt, sl] = jnp.minimum(mnv[j], xs[j])
                dbase = (dt // 16) * 16
                dwin = acc_deg[pl.ds(dbase, 16)]
                acc_deg[pl.ds(dbase, 16)] = dwin + jnp.where(
                    lane == dt - dbase, 1.0, 0.0)
                return 0

            lax.fori_loop(0, n_edges, edge_body, 0)
            pltpu.sync_copy(xbuf, ssum.at[gidx_d], add=True)
            pltpu.sync_copy(x2buf, ssq.at[gidx_d], add=True)

        def init_body(r, _):
            for j in range(8):
                sl = pl.ds(j * 16, 16)
                acc_mx[r, sl] = zeros16 - FLT_MAX
                acc_mn[r, sl] = zeros16 + FLT_MAX
            return 0
        lax.fori_loop(0, NPG, init_body, 0)

        def dinit_body(r, _):
            acc_deg[pl.ds(r * 16, 16)] = zeros16
            return 0
        lax.fori_loop(0, NPG // 16, dinit_body, 0)

        def zinit_body(r, _):
            for j in range(8):
                xbuf[r, pl.ds(j * 16, 16)] = zeros16
            return 0
        lax.fori_loop(0, FK, zinit_body, 0)
        for q in range(NPG // 64):
            sl = pl.ds(tlo + q * 64, 64)
            pltpu.sync_copy(xbuf.at[pl.ds(0, 64)], ssum.at[sl])
            pltpu.sync_copy(xbuf.at[pl.ds(0, 64)], ssq.at[sl])
        if NPG % 64:
            sl = pl.ds(tlo + (NPG // 64) * 64, NPG % 64)
            pltpu.sync_copy(xbuf.at[pl.ds(0, NPG % 64)], ssum.at[sl])
            pltpu.sync_copy(xbuf.at[pl.ds(0, NPG % 64)], ssq.at[sl])

        @pl.when(s == 0)
        def _():
            dsl = pl.ds(NPSC, 8)
            pltpu.sync_copy(xbuf.at[pl.ds(0, 8)], ssum.at[dsl])
            pltpu.sync_copy(xbuf.at[pl.ds(0, 8)], ssq.at[dsl])

        for half in range(0, FK + 64, 16):
            pend_d[pl.ds(half, 16)] = izeros16
            pend_s[pl.ds(half, 16)] = izeros16
            pend_e[pl.ds(half, 16)] = izeros16

        plsc.subcore_barrier()

        def scan_chunk(off, dbuf, sbuf, cnt):
            def filt_body(i, cnt):
                for u in range(4):
                    sl16 = pl.ds(i * 64 + u * 16, 16)
                    dv = dbuf[sl16]
                    m = (dv >= glo) & (dv < glo + NPG)
                    nm = plsc.all_reduce_population_count(m)[0]
                    plsc.store_compressed(pend_d.at[pl.ds(cnt, 16)],
                                          dv - sclo, mask=m)
                    plsc.store_compressed(pend_s.at[pl.ds(cnt, 16)],
                                          sbuf[sl16], mask=m)
                    plsc.store_compressed(pend_e.at[pl.ds(cnt, 16)],
                                          off + i * 64 + u * 16 + lane,
                                          mask=m)
                    cnt = cnt + nm
                do_flush = cnt >= FK

                @pl.when(do_flush)
                def _():
                    process_edges(FK)
                    for half in (0, 16, 32, 48):
                        mv = pl.ds(half, 16)
                        mv2 = pl.ds(FK + half, 16)
                        pend_d[mv] = pend_d[mv2]
                        pend_s[mv] = pend_s[mv2]
                        pend_e[mv] = pend_e[mv2]

                return jnp.where(do_flush, cnt - FK, cnt)

            return lax.fori_loop(0, NITER // 4, filt_body, cnt)

        def start_load(ci, dbuf, sbuf, semd, sems):
            off = ci * CHUNK
            pltpu.async_copy(dst_hbm.at[pl.ds(off, CHUNK)], dbuf, semd)
            pltpu.async_copy(src_hbm.at[pl.ds(off, CHUNK)], sbuf, sems)

        def wait_load(dbuf, sbuf, semd, sems):
            pltpu.make_async_copy(dst_hbm.at[pl.ds(0, CHUNK)], dbuf,
                                  semd).wait()
            pltpu.make_async_copy(src_hbm.at[pl.ds(0, CHUNK)], sbuf,
                                  sems).wait()

        start_load(0, dbuf0, sbuf0, semd0, sems0)

        def pair_body(q, cnt):
            start_load(2 * q + 1, dbuf1, sbuf1, semd1, sems1)
            wait_load(dbuf0, sbuf0, semd0, sems0)
            cnt = scan_chunk(2 * q * CHUNK, dbuf0, sbuf0, cnt)

            @pl.when(q < NCHUNK // 2 - 1)
            def _():
                start_load(2 * q + 2, dbuf0, sbuf0, semd0, sems0)

            wait_load(dbuf1, sbuf1, semd1, sems1)
            return scan_chunk((2 * q + 1) * CHUNK, dbuf1, sbuf1, cnt)

        cnt = lax.fori_loop(0, NCHUNK // 2, pair_body, jnp.int32(0))

        @pl.when(cnt > 0)
        def _():
            process_edges(cnt)

        plsc.subcore_barrier()

        pltpu.sync_copy(ssum.at[pl.ds(tlo, NPG)], o_sum.at[pl.ds(glo, NPG)])
        pltpu.sync_copy(ssq.at[pl.ds(tlo, NPG)], o_sq.at[pl.ds(glo, NPG)])
        pltpu.sync_copy(acc_mx, o_mx.at[pl.ds(glo, NPG)])
        pltpu.sync_copy(acc_mn, o_mn.at[pl.ds(glo, NPG)])
        pltpu.sync_copy(acc_deg, o_deg.at[pl.ds(glo, NPG)])

    for p in range(NPASS):
        g = p * NW + c * NS + s
        run_pass(g * NPG, (p * NW + c * NS) * NPG)
        if p + 1 < NPASS:
            plsc.subcore_barrier()


def _sc_aggregate(A, C, src, dst):
    f32 = jnp.float32
    i32 = jnp.int32
    out_type = (jax.ShapeDtypeStruct((NPAD, D), f32),
                jax.ShapeDtypeStruct((NPAD, D), f32),
                jax.ShapeDtypeStruct((NPAD, D), f32),
                jax.ShapeDtypeStruct((NPAD, D), f32),
                jax.ShapeDtypeStruct((NPAD,), f32))
    scratch = [
        pltpu.VMEM_SHARED((NPSC + 8, D), f32),
        pltpu.VMEM_SHARED((NPSC + 8, D), f32),
        pltpu.VMEM((NPG, D), f32),
        pltpu.VMEM((NPG, D), f32),
        pltpu.VMEM((NPG,), f32),
        pltpu.VMEM((CHUNK,), i32),
        pltpu.VMEM((CHUNK,), i32),
        pltpu.VMEM((CHUNK,), i32),
        pltpu.VMEM((CHUNK,), i32),
        pltpu.VMEM((FK + 64,), i32),
        pltpu.VMEM((FK + 64,), i32),
        pltpu.VMEM((FK + 64,), i32),
        pltpu.VMEM((FK,), i32),
        pltpu.VMEM((FK,), i32),
        pltpu.VMEM((FK,), i32),
        pltpu.VMEM((FK, D), f32),
        pltpu.VMEM((FK, D), f32),
        pltpu.VMEM((FK, D), f32),
        pltpu.VMEM((FK, D), f32),
        pltpu.SemaphoreType.DMA,
        pltpu.SemaphoreType.DMA,
        pltpu.SemaphoreType.DMA,
        pltpu.SemaphoreType.DMA,
        pltpu.SemaphoreType.DMA,
        pltpu.SemaphoreType.DMA,
    ]
    mesh = plsc.VectorSubcoreMesh(core_axis_name="c", subcore_axis_name="s",
                                  num_cores=NC, num_subcores=NS)
    fn = pl.kernel(_sc_body, out_type=out_type, mesh=mesh,
                   scratch_types=scratch,
                   compiler_params=pltpu.CompilerParams(
                       needs_layout_passes=False))
    return fn(A, C, dst, src)


def _post_body(h_ref, s1_ref, s2_ref, mx_ref, mn_ref, deg_ref, sn_ref,
               wpd_ref, wh_ref, wa_ref, wb_ref, wc_ref, bp_ref, o_ref):
    f32 = jnp.float32
    h = h_ref[...]
    B = jnp.dot(h, wpd_ref[...], preferred_element_type=f32)
    deg = deg_ref[...]
    s1 = s1_ref[...]
    s2 = s2_ref[...]
    degc = jnp.maximum(deg, 1.0)
    has = deg > 0.0
    mean = (s1 + deg * B) / degc
    mx = jnp.where(has, mx_ref[...] + B, 0.0)
    mn = jnp.where(has, mn_ref[...] + B, 0.0)
    mean_sq = (s2 + 2.0 * B * s1 + deg * B * B) / degc
    var = jnp.maximum(mean_sq - mean * mean, 0.0)
    std = jnp.sqrt(var + EPS)
    agg = jnp.concatenate([mean, mx, mn, std], axis=1)
    logd = jnp.log(degc + 1.0)
    amp = logd * (1.0 / AVG_D_LOG)
    att = AVG_D_LOG / logd
    acc = (jnp.dot(h, wh_ref[...], preferred_element_type=f32)
           + jnp.dot(agg, wa_ref[...], preferred_element_type=f32)
           + jnp.dot(agg * amp, wb_ref[...], preferred_element_type=f32)
           + jnp.dot(agg * att, wc_ref[...], preferred_element_type=f32))
    o_ref[...] = (acc + bp_ref[...]) * sn_ref[...]


def _tc_post(h, s1, s2, mx, mn, deg, snorm, Wpd, Wh, Wa, Wb, Wc, b_post):
    R = 2000
    node_spec = pl.BlockSpec((R, D), lambda i: (i, 0))
    col_spec = pl.BlockSpec((R, 1), lambda i: (i, 0))
    full = lambda r, c: pl.BlockSpec((r, c), lambda i: (0, 0))
    return pl.pallas_call(
        _post_body,
        grid=(5,),
        in_specs=[node_spec, node_spec, node_spec, node_spec, node_spec,
                  col_spec, col_spec,
                  full(D, D), full(D, D), full(512, D), full(512, D),
                  full(512, D), full(1, D)],
        out_specs=node_spec,
        out_shape=jax.ShapeDtypeStruct((N, D), jnp.float32),
    )(h, s1, s2, mx, mn, deg, snorm, Wpd, Wh, Wa, Wb, Wc, b_post)


def kernel(h, edge_index, e, snorm_n, W_pre, b_pre, W_post, b_post):
    Wps = W_pre[0:D]
    Wpd = W_pre[D:2 * D]
    Wpe = W_pre[2 * D:]
    Wh = W_post[0:D]
    Wa = W_post[D:D + 512]
    Wb = W_post[D + 512:D + 1024]
    Wc = W_post[D + 1024:]
    src = edge_index[0]
    dst = edge_index[1]

    A, C = _tc_pre(h, e, Wps, Wpe, b_pre)
    s1, s2, mx, mn, deg = _sc_aggregate(A, C, src, dst)
    out = _tc_post(h, s1[:N], s2[:N], mx[:N], mn[:N],
                   deg[:N].reshape(N, 1), snorm_n,
                   Wpd, Wh, Wa, Wb, Wc, b_post.reshape(1, D))
    return out
